# baseline (speedup 1.0000x reference)
import jax
import jax.numpy as jnp
from jax.experimental import pallas as pl
from jax.experimental.pallas import tpu as pltpu

ROWS_PER_BLOCK = 256
CHUNK = 128


def _cumsum_kernel(x_ref, o_ref):
    ncols = x_ref.shape[1]
    nchunks = ncols // CHUNK
    row_i = jax.lax.broadcasted_iota(jnp.int32, (CHUNK, CHUNK), 0)
    col_i = jax.lax.broadcasted_iota(jnp.int32, (CHUNK, CHUNK), 1)
    tri = (row_i <= col_i).astype(jnp.float32)

    totals = []
    for c in range(nchunks):
        xc = x_ref[:, c * CHUNK:(c + 1) * CHUNK]
        local = jax.lax.dot(xc, tri, preferred_element_type=jnp.float32)
        o_ref[:, c * CHUNK:(c + 1) * CHUNK] = local
        totals.append(local[:, CHUNK - 1:CHUNK])

    t = jnp.concatenate(totals, axis=1)
    g_row = jax.lax.broadcasted_iota(jnp.int32, (nchunks, nchunks), 0)
    g_col = jax.lax.broadcasted_iota(jnp.int32, (nchunks, nchunks), 1)
    strict = (g_row < g_col).astype(jnp.float32)
    carries = jax.lax.dot(t, strict, preferred_element_type=jnp.float32)

    for c in range(nchunks):
        sl = slice(c * CHUNK, (c + 1) * CHUNK)
        o_ref[:, sl] = o_ref[:, sl] + carries[:, c:c + 1]


def kernel(x):
    m, n = x.shape
    return pl.pallas_call(
        _cumsum_kernel,
        grid=(m // ROWS_PER_BLOCK,),
        in_specs=[pl.BlockSpec((ROWS_PER_BLOCK, n), lambda i: (i, 0))],
        out_specs=pl.BlockSpec((ROWS_PER_BLOCK, n), lambda i: (i, 0)),
        out_shape=jax.ShapeDtypeStruct((m, n), x.dtype),
        compiler_params=pltpu.CompilerParams(
            dimension_semantics=("parallel",)),
    )(x)


# speedup vs baseline: 1.1269x; 1.1269x over previous
import jax
import jax.numpy as jnp
from jax.experimental import pallas as pl
from jax.experimental.pallas import tpu as pltpu

ROWS_PER_BLOCK = 256
CHUNK = 128


def _cumsum_kernel(x_ref, o_ref):
    rows = x_ref.shape[0]
    ncols = x_ref.shape[1]
    nchunks = ncols // CHUNK
    row_i = jax.lax.broadcasted_iota(jnp.int32, (CHUNK, CHUNK), 0)
    col_i = jax.lax.broadcasted_iota(jnp.int32, (CHUNK, CHUNK), 1)
    tri = (row_i <= col_i).astype(jnp.float32)

    carry = jnp.zeros((rows, 1), jnp.float32)
    for c in range(nchunks):
        xc = x_ref[:, c * CHUNK:(c + 1) * CHUNK]
        local = jax.lax.dot(xc, tri, preferred_element_type=jnp.float32)
        o_ref[:, c * CHUNK:(c + 1) * CHUNK] = local + carry
        carry = carry + local[:, CHUNK - 1:CHUNK]


def kernel(x):
    m, n = x.shape
    return pl.pallas_call(
        _cumsum_kernel,
        grid=(m // ROWS_PER_BLOCK,),
        in_specs=[pl.BlockSpec((ROWS_PER_BLOCK, n), lambda i: (i, 0))],
        out_specs=pl.BlockSpec((ROWS_PER_BLOCK, n), lambda i: (i, 0)),
        out_shape=jax.ShapeDtypeStruct((m, n), x.dtype),
        compiler_params=pltpu.CompilerParams(
            dimension_semantics=("parallel",)),
    )(x)


# speedup vs baseline: 1.1643x; 1.0332x over previous
import jax
import jax.numpy as jnp
from jax.experimental import pallas as pl
from jax.experimental.pallas import tpu as pltpu

ROWS_PER_BLOCK = 256
CHUNK = 256


def _cumsum_kernel(x_ref, o_ref):
    rows = x_ref.shape[0]
    ncols = x_ref.shape[1]
    nchunks = ncols // CHUNK
    row_i = jax.lax.broadcasted_iota(jnp.int32, (CHUNK, CHUNK), 0)
    col_i = jax.lax.broadcasted_iota(jnp.int32, (CHUNK, CHUNK), 1)
    tri = (row_i <= col_i).astype(jnp.float32)

    carry = jnp.zeros((rows, 1), jnp.float32)
    for c in range(nchunks):
        xc = x_ref[:, c * CHUNK:(c + 1) * CHUNK]
        local = jax.lax.dot(xc, tri, preferred_element_type=jnp.float32)
        o_ref[:, c * CHUNK:(c + 1) * CHUNK] = local + carry
        carry = carry + local[:, CHUNK - 1:CHUNK]


def kernel(x):
    m, n = x.shape
    return pl.pallas_call(
        _cumsum_kernel,
        grid=(m // ROWS_PER_BLOCK,),
        in_specs=[pl.BlockSpec((ROWS_PER_BLOCK, n), lambda i: (i, 0))],
        out_specs=pl.BlockSpec((ROWS_PER_BLOCK, n), lambda i: (i, 0)),
        out_shape=jax.ShapeDtypeStruct((m, n), x.dtype),
        compiler_params=pltpu.CompilerParams(
            dimension_semantics=("parallel",)),
    )(x)
